# Initial kernel scaffold; baseline (speedup 1.0000x reference)
import jax
import jax.numpy as jnp
from jax.experimental import pallas as pl


def kernel(x, flow, sample_grid):
    raise NotImplementedError("write your pallas kernel here")


# speedup vs baseline: 1.6619x; 1.6619x over previous
import functools

import jax
import jax.numpy as jnp
from jax import lax
from jax.experimental import pallas as pl
from jax.experimental.pallas import tpu as pltpu
from jax.experimental.pallas import tpu_sc as plsc

N, C, D, H, W = 2, 3, 128, 128, 128
LOCS = N * D * H * W
VOL = D * H * W
NW = 32
PER_W = LOCS // NW
CHUNK = 4096
NCHUNK = PER_W // CHUNK
ROWS = CHUNK // 128

_MAGIC = 12582912.0


def _body(xflat, fx_h, fy_h, fz_h, sx_h, sy_h, sz_h, out_h,
          fx_v, fy_v, fz_v, sx_v, sy_v, sz_v, idx_v, vals_v, sem_in, sem_g):
    c_id = lax.axis_index("c")
    s_id = lax.axis_index("s")

    hbm_ins = (fx_h, fy_h, fz_h, sx_h, sy_h, sz_h)
    vmem_ins = (fx_v, fy_v, fz_v, sx_v, sy_v, sz_v)

    @pl.loop(0, NCHUNK)
    def _chunk(k):
        base = c_id * VOL + s_id * PER_W + k * CHUNK
        descs = [pltpu.async_copy(h.at[pl.ds(base, CHUNK)], v, sem_in)
                 for h, v in zip(hbm_ins, vmem_ins)]
        for dd in descs:
            dd.wait()

        off0 = c_id * (C * VOL)

        @pl.loop(0, ROWS)
        def _compute(j):
            for l in range(8):
                s_ = pl.ds(j * 128 + l * 16, 16)

                def to_idx(g, size):
                    t = ((g + 1.0) * 0.5) * float(size - 1)
                    r = (t + _MAGIC) - _MAGIC
                    r = jnp.minimum(jnp.maximum(r, 0.0), float(size - 1))
                    return r.astype(jnp.int32)

                ix = to_idx(fx_v[s_] + sx_v[s_], W)
                iy = to_idx(fy_v[s_] + sy_v[s_], H)
                iz = to_idx(fz_v[s_] + sz_v[s_], D)
                flat = (iz * (H * W) + iy * W + ix) + off0
                for ch in range(C):
                    idx_v[ch, j, pl.ds(l * 16, 16)] = flat + ch * VOL

        @pl.loop(0, ROWS)
        def _fire(j):
            for ch in range(C):
                pltpu.async_copy(xflat.at[idx_v.at[ch, j]],
                                 vals_v.at[ch, j], sem_g)

        @pl.loop(0, ROWS)
        def _drain(j):
            for ch in range(C):
                pltpu.make_async_copy(xflat.at[idx_v.at[ch, j]],
                                      vals_v.at[ch, j], sem_g).wait()

        for ch in range(C):
            row_off = (c_id * (C * VOL) + ch * VOL + s_id * PER_W
                       + k * CHUNK) // 128
            row_off = pl.multiple_of(row_off, 8)
            pltpu.sync_copy(vals_v.at[ch], out_h.at[pl.ds(row_off, ROWS)])


_grid_sample_sc = pl.kernel(
    _body,
    out_type=jax.ShapeDtypeStruct((N * C * VOL // 128, 128), jnp.float32),
    mesh=plsc.VectorSubcoreMesh(core_axis_name="c", subcore_axis_name="s"),
    scratch_types=[
        pltpu.VMEM((CHUNK,), jnp.float32),
        pltpu.VMEM((CHUNK,), jnp.float32),
        pltpu.VMEM((CHUNK,), jnp.float32),
        pltpu.VMEM((CHUNK,), jnp.float32),
        pltpu.VMEM((CHUNK,), jnp.float32),
        pltpu.VMEM((CHUNK,), jnp.float32),
        pltpu.VMEM((C, ROWS, 128), jnp.int32),
        pltpu.VMEM((C, ROWS, 128), jnp.float32),
        pltpu.SemaphoreType.DMA,
        pltpu.SemaphoreType.DMA,
    ],
)


def kernel(x, flow, sample_grid):
    assert x.shape == (N, C, D, H, W)
    xflat = x.reshape(-1)
    fx = flow[..., 0].reshape(-1)
    fy = flow[..., 1].reshape(-1)
    fz = flow[..., 2].reshape(-1)
    sx = sample_grid[..., 0].reshape(-1)
    sy = sample_grid[..., 1].reshape(-1)
    sz = sample_grid[..., 2].reshape(-1)
    out = _grid_sample_sc(xflat, fx, fy, fz, sx, sy, sz)
    return out.reshape(N, C, D, H, W)
